# Initial kernel scaffold; baseline (speedup 1.0000x reference)
import jax
import jax.numpy as jnp
from jax.experimental import pallas as pl


def kernel(features, edge_index, W1, b1, W2, b2, W3, b3, Wm, bm):
    raise NotImplementedError("write your pallas kernel here")


# speedup vs baseline: 2.4547x; 2.4547x over previous
import functools

import jax
import jax.numpy as jnp
from jax import lax
from jax.experimental import pallas as pl
from jax.experimental.pallas import tpu as pltpu
from jax.experimental.pallas import tpu_sc as plsc

N = 10000
E = 160000
D_IN = 256
D_H = 512
D_OUT = 256

NS = 16
NC = 2
EPT = E // NS
B_E = 80
NB_E = EPT // B_E
N_PAD = 10240
STRIPE = N_PAD // NS
NCH = D_H // 128

HALF_N = 5120
ACC_R = 5376
ZST = ACC_R // NS
DST = HALF_N // NS

N_BLK = 1000
GRID = N // N_BLK


@functools.lru_cache(maxsize=None)
def _sc_mesh():
    return plsc.VectorSubcoreMesh(core_axis_name="c", subcore_axis_name="s")


_SC_PARAMS = pltpu.CompilerParams(needs_layout_passes=False)


@functools.lru_cache(maxsize=None)
def _build_deg():
    @functools.partial(
        pl.kernel,
        out_type=[jax.ShapeDtypeStruct((N_PAD,), jnp.float32)] * 2,
        mesh=_sc_mesh(),
        compiler_params=_SC_PARAMS,
        scratch_types=[
            pltpu.VMEM((EPT,), jnp.int32),
            pltpu.VMEM((N_PAD,), jnp.float32),
            pltpu.VMEM((NS, STRIPE), jnp.float32),
            pltpu.VMEM((STRIPE,), jnp.float32),
            pltpu.VMEM_SHARED((NS, N_PAD), jnp.float32),
        ],
    )
    def deg_kernel(eidx, out_src, out_dst, idx_buf, cnt, buf, stripe_buf,
                   shared):
        c = lax.axis_index("c")
        s = lax.axis_index("s")
        pltpu.sync_copy(eidx.at[c, s], idx_buf)
        zeros16 = jnp.zeros((16,), jnp.float32)

        def zero_body(i, carry):
            cnt[pl.ds(i * 16, 16)] = zeros16
            return carry

        lax.fori_loop(0, N_PAD // 16, zero_body, 0)
        ones16 = jnp.ones((16,), jnp.float32)

        def cnt_body(i, carry):
            v = idx_buf[pl.ds(i * 16, 16)]
            plsc.addupdate_scatter(cnt, [v], ones16)
            return carry

        lax.fori_loop(0, EPT // 16, cnt_body, 0)
        pltpu.sync_copy(cnt, shared.at[s])
        plsc.subcore_barrier()
        off = s * STRIPE
        pltpu.sync_copy(shared.at[:, pl.ds(off, STRIPE)], buf)

        def red_body(k, carry):
            acc = zeros16
            for t in range(NS):
                acc = acc + buf[t, pl.ds(k * 16, 16)]
            stripe_buf[pl.ds(k * 16, 16)] = acc
            return carry

        lax.fori_loop(0, STRIPE // 16, red_body, 0)

        @pl.when(c == 0)
        def _():
            pltpu.sync_copy(stripe_buf, out_src.at[pl.ds(off, STRIPE)])

        @pl.when(c == 1)
        def _():
            pltpu.sync_copy(stripe_buf, out_dst.at[pl.ds(off, STRIPE)])

    return deg_kernel


@functools.lru_cache(maxsize=None)
def _build_agg():
    @functools.partial(
        pl.kernel,
        out_type=[jax.ShapeDtypeStruct((N_PAD, 128), jnp.float32)] * NCH,
        mesh=_sc_mesh(),
        compiler_params=_SC_PARAMS,
        scratch_types=[
            pltpu.VMEM((NB_E, B_E), jnp.int32),
            pltpu.VMEM((NB_E, B_E), jnp.int32),
            pltpu.VMEM((NB_E, B_E), jnp.int32),
            pltpu.VMEM((B_E, 128), jnp.float32),
            pltpu.VMEM((48, 128), jnp.float32),
            pltpu.VMEM((80, 128), jnp.float32),
            pltpu.VMEM_SHARED((ACC_R, 128), jnp.float32),
            pltpu.SemaphoreType.DMA,
        ],
    )
    def agg_kernel(h0, h1, h2, h3, srcr, dstr,
                   o0, o1, o2, o3,
                   src_buf, dst_buf, adj_buf, rows, zero_buf, drain_buf,
                   acc, sem):
        c = lax.axis_index("c")
        s = lax.axis_index("s")
        pltpu.sync_copy(srcr.at[s], src_buf)
        pltpu.sync_copy(dstr.at[s], dst_buf)
        zeros16 = jnp.zeros((16,), jnp.float32)

        def zb(i, carry):
            for k in range(128 // 16):
                zero_buf[i, pl.ds(k * 16, 16)] = zeros16
            return carry

        lax.fori_loop(0, 48, zb, 0)
        dump_vec = jnp.zeros((16,), jnp.int32) + (HALF_N + (s % 8))
        h_refs = (h0, h1, h2, h3)
        o_refs = (o0, o1, o2, o3)
        for chunk in range(NCH):

            @pl.when(c == chunk // 2)
            def _(chunk=chunk):
                h_ref = h_refs[chunk]
                o_ref = o_refs[chunk]
                for p in range(2):
                    lo = p * HALF_N

                    def adj(r, carry):
                        for k in range(B_E // 16):
                            d = dst_buf[r, pl.ds(k * 16, 16)] - lo
                            ok = (d >= 0) & (d < HALF_N)
                            adj_buf[r, pl.ds(k * 16, 16)] = jnp.where(
                                ok, d, dump_vec)
                        return carry

                    lax.fori_loop(0, NB_E, adj, 0)
                    zbase = s * ZST
                    for i in range(ZST // 48):
                        pltpu.sync_copy(
                            zero_buf, acc.at[pl.ds(zbase + i * 48, 48)])
                    plsc.subcore_barrier()

                    def eb(j, carry):
                        pltpu.async_copy(h_ref.at[src_buf.at[j]], rows,
                                         sem).wait()
                        pltpu.sync_copy(rows, acc.at[adj_buf.at[j]], add=True)
                        return carry

                    lax.fori_loop(0, NB_E, eb, 0)
                    plsc.subcore_barrier()
                    dbase = s * DST
                    for i in range(DST // 80):
                        pltpu.sync_copy(acc.at[pl.ds(dbase + i * 80, 80)],
                                        drain_buf)
                        pltpu.sync_copy(
                            drain_buf,
                            o_ref.at[pl.ds(lo + dbase + i * 80, 80)])
                    plsc.subcore_barrier()

    return agg_kernel


@functools.lru_cache(maxsize=None)
def _build_b1():
    def body(deg_ref, x_ref, w_ref, h0, h1, h2, h3, ns_ref, nd_ref):
        d = deg_ref[...]
        ns = lax.rsqrt(jnp.maximum(d[:, 0:1], 1.0))
        nd = lax.rsqrt(jnp.maximum(d[:, 1:2], 1.0))
        h = jnp.dot(x_ref[...], w_ref[...],
                    preferred_element_type=jnp.float32) * ns
        for k, r in enumerate((h0, h1, h2, h3)):
            r[...] = h[:, k * 128:(k + 1) * 128]
        ns_ref[...] = ns
        nd_ref[...] = nd

    return pl.pallas_call(
        body,
        grid=(GRID,),
        in_specs=[
            pl.BlockSpec((N_BLK, 2), lambda i: (i, 0)),
            pl.BlockSpec((N_BLK, D_IN), lambda i: (i, 0)),
            pl.BlockSpec((D_IN, D_H), lambda i: (0, 0)),
        ],
        out_specs=[pl.BlockSpec((N_BLK, 128), lambda i: (i, 0))] * NCH
        + [pl.BlockSpec((N_BLK, 1), lambda i: (i, 0))] * 2,
        out_shape=[jax.ShapeDtypeStruct((N, 128), jnp.float32)] * NCH
        + [jax.ShapeDtypeStruct((N, 1), jnp.float32)] * 2,
    )


@functools.lru_cache(maxsize=None)
def _build_b23():
    def body(a0, a1, a2, a3, nd_ref, b_ref, ns_ref, w_ref, h0, h1, h2, h3):
        x = jnp.concatenate([a0[...], a1[...], a2[...], a3[...]], axis=1)
        x = jnp.maximum(x * nd_ref[...] + b_ref[...], 0.0)
        h = jnp.dot(x, w_ref[...],
                    preferred_element_type=jnp.float32) * ns_ref[...]
        for k, r in enumerate((h0, h1, h2, h3)):
            r[...] = h[:, k * 128:(k + 1) * 128]

    return pl.pallas_call(
        body,
        grid=(GRID,),
        in_specs=[pl.BlockSpec((N_BLK, 128), lambda i: (i, 0))] * NCH
        + [
            pl.BlockSpec((N_BLK, 1), lambda i: (i, 0)),
            pl.BlockSpec((1, D_H), lambda i: (0, 0)),
            pl.BlockSpec((N_BLK, 1), lambda i: (i, 0)),
            pl.BlockSpec((D_H, D_H), lambda i: (0, 0)),
        ],
        out_specs=[pl.BlockSpec((N_BLK, 128), lambda i: (i, 0))] * NCH,
        out_shape=[jax.ShapeDtypeStruct((N, 128), jnp.float32)] * NCH,
    )


@functools.lru_cache(maxsize=None)
def _build_b4():
    def body(a0, a1, a2, a3, nd_ref, b_ref, wm_ref, bm_ref, out_ref):
        x = jnp.concatenate([a0[...], a1[...], a2[...], a3[...]], axis=1)
        x = x * nd_ref[...] + b_ref[...]
        out_ref[...] = jnp.dot(x, wm_ref[...],
                               preferred_element_type=jnp.float32) + bm_ref[...]

    return pl.pallas_call(
        body,
        grid=(GRID,),
        in_specs=[pl.BlockSpec((N_BLK, 128), lambda i: (i, 0))] * NCH
        + [
            pl.BlockSpec((N_BLK, 1), lambda i: (i, 0)),
            pl.BlockSpec((1, D_H), lambda i: (0, 0)),
            pl.BlockSpec((D_H, D_OUT), lambda i: (0, 0)),
            pl.BlockSpec((1, D_OUT), lambda i: (0, 0)),
        ],
        out_specs=pl.BlockSpec((N_BLK, D_OUT), lambda i: (i, 0)),
        out_shape=jax.ShapeDtypeStruct((N, D_OUT), jnp.float32),
    )


def kernel(features, edge_index, W1, b1, W2, b2, W3, b3, Wm, bm):
    src = edge_index[0].reshape(NS, NB_E, B_E)
    dst = edge_index[1].reshape(NS, NB_E, B_E)
    eidx = edge_index.reshape(2, NS, EPT)

    deg_out, deg_in = _build_deg()(eidx)
    degT = jnp.stack([deg_out[:N], deg_in[:N]], axis=1)

    h0, h1, h2, h3, ns, nd = _build_b1()(degT, features, W1)
    agg = _build_agg()
    b23 = _build_b23()

    a = agg(h0, h1, h2, h3, src, dst)
    h = b23(*a, nd, b1.reshape(1, D_H), ns, W2)
    a = agg(*h, src, dst)
    h = b23(*a, nd, b2.reshape(1, D_H), ns, W3)
    a = agg(*h, src, dst)
    return _build_b4()(*a, nd, b3.reshape(1, D_H), Wm, bm.reshape(1, D_OUT))


# speedup vs baseline: 3.2686x; 1.3316x over previous
import functools

import jax
import jax.numpy as jnp
from jax import lax
from jax.experimental import pallas as pl
from jax.experimental.pallas import tpu as pltpu
from jax.experimental.pallas import tpu_sc as plsc

N = 10000
E = 160000
D_IN = 256
D_H = 512
D_OUT = 256

NS = 16
NC = 2
EPT = E // NS
B_E = 80
NB_E = EPT // B_E
N_PAD = 10240
STRIPE = N_PAD // NS
NCH = D_H // 128

HALF_N = 5120
ACC_R = 6400
ZST = ACC_R // NS
DST = HALF_N // NS

N_BLK = 1000
GRID = N // N_BLK


@functools.lru_cache(maxsize=None)
def _sc_mesh():
    return plsc.VectorSubcoreMesh(core_axis_name="c", subcore_axis_name="s")


_SC_PARAMS = pltpu.CompilerParams(needs_layout_passes=False)


@functools.lru_cache(maxsize=None)
def _build_deg():
    @functools.partial(
        pl.kernel,
        out_type=[jax.ShapeDtypeStruct((N_PAD,), jnp.float32)] * 2,
        mesh=_sc_mesh(),
        compiler_params=_SC_PARAMS,
        scratch_types=[
            pltpu.VMEM((EPT,), jnp.int32),
            pltpu.VMEM((N_PAD,), jnp.float32),
            pltpu.VMEM((NS, STRIPE), jnp.float32),
            pltpu.VMEM((STRIPE,), jnp.float32),
            pltpu.VMEM_SHARED((NS, N_PAD), jnp.float32),
        ],
    )
    def deg_kernel(eidx, out_src, out_dst, idx_buf, cnt, buf, stripe_buf,
                   shared):
        c = lax.axis_index("c")
        s = lax.axis_index("s")
        pltpu.sync_copy(eidx.at[c, s], idx_buf)
        zeros16 = jnp.zeros((16,), jnp.float32)

        def zero_body(i, carry):
            cnt[pl.ds(i * 16, 16)] = zeros16
            return carry

        lax.fori_loop(0, N_PAD // 16, zero_body, 0)
        ones16 = jnp.ones((16,), jnp.float32)

        def cnt_body(i, carry):
            v = idx_buf[pl.ds(i * 16, 16)]
            plsc.addupdate_scatter(cnt, [v], ones16)
            return carry

        lax.fori_loop(0, EPT // 16, cnt_body, 0)
        pltpu.sync_copy(cnt, shared.at[s])
        plsc.subcore_barrier()
        off = s * STRIPE
        pltpu.sync_copy(shared.at[:, pl.ds(off, STRIPE)], buf)

        def red_body(k, carry):
            acc = zeros16
            for t in range(NS):
                acc = acc + buf[t, pl.ds(k * 16, 16)]
            stripe_buf[pl.ds(k * 16, 16)] = acc
            return carry

        lax.fori_loop(0, STRIPE // 16, red_body, 0)

        @pl.when(c == 0)
        def _():
            pltpu.sync_copy(stripe_buf, out_src.at[pl.ds(off, STRIPE)])

        @pl.when(c == 1)
        def _():
            pltpu.sync_copy(stripe_buf, out_dst.at[pl.ds(off, STRIPE)])

    return deg_kernel


@functools.lru_cache(maxsize=None)
def _build_agg():
    @functools.partial(
        pl.kernel,
        out_type=[jax.ShapeDtypeStruct((N_PAD, 128), jnp.float32)] * NCH,
        mesh=_sc_mesh(),
        compiler_params=_SC_PARAMS,
        scratch_types=[
            pltpu.VMEM((NB_E, B_E), jnp.int32),
            pltpu.VMEM((NB_E, B_E), jnp.int32),
            pltpu.VMEM((NB_E, B_E), jnp.int32),
            pltpu.VMEM((B_E, 128), jnp.float32),
            pltpu.VMEM((B_E, 128), jnp.float32),
            pltpu.VMEM_SHARED((ACC_R, 128), jnp.float32),
            pltpu.SemaphoreType.DMA,
            pltpu.SemaphoreType.DMA,
            pltpu.SemaphoreType.DMA,
            pltpu.SemaphoreType.DMA,
        ],
    )
    def agg_kernel(h0, h1, h2, h3, srcr, dstr,
                   o0, o1, o2, o3,
                   src_buf, dst_buf, adj_buf, buf0, buf1, acc,
                   sg0, sg1, ss0, ss1):
        c = lax.axis_index("c")
        s = lax.axis_index("s")
        pltpu.sync_copy(srcr.at[s], src_buf)
        pltpu.sync_copy(dstr.at[s], dst_buf)
        zeros16 = jnp.zeros((16,), jnp.float32)
        dump_vec = jnp.zeros((16,), jnp.int32) + (HALF_N + (s % 8))
        h_refs = (h0, h1, h2, h3)
        o_refs = (o0, o1, o2, o3)

        def wait_g(buf, sem, h_ref):
            pltpu.make_async_copy(h_ref.at[src_buf.at[0]], buf, sem).wait()

        def wait_s(buf, sem):
            pltpu.make_async_copy(buf, acc.at[adj_buf.at[0]], sem).wait()

        for chunk in range(NCH):

            @pl.when(c == chunk // 2)
            def _(chunk=chunk):
                h_ref = h_refs[chunk]
                o_ref = o_refs[chunk]
                for p in range(2):
                    lo = p * HALF_N

                    def adj(r, carry):
                        for k in range(B_E // 16):
                            d = dst_buf[r, pl.ds(k * 16, 16)] - lo
                            ok = (d >= 0) & (d < HALF_N)
                            adj_buf[r, pl.ds(k * 16, 16)] = jnp.where(
                                ok, d, dump_vec)
                        return carry

                    lax.fori_loop(0, NB_E, adj, 0)

                    def zb(i, carry):
                        for k in range(128 // 16):
                            buf0[i, pl.ds(k * 16, 16)] = zeros16
                        return carry

                    lax.fori_loop(0, B_E, zb, 0)
                    zbase = s * ZST
                    for i in range(ZST // B_E):
                        pltpu.sync_copy(buf0,
                                        acc.at[pl.ds(zbase + i * B_E, B_E)])
                    plsc.subcore_barrier()

                    pltpu.async_copy(h_ref.at[src_buf.at[0]], buf0, sg0)
                    pltpu.async_copy(h_ref.at[src_buf.at[1]], buf1, sg1)

                    def eb(i, carry):
                        j0 = 2 * i
                        j1 = 2 * i + 1
                        wait_g(buf0, sg0, h_ref)
                        pltpu.async_copy(buf0, acc.at[adj_buf.at[j0]], ss0,
                                         add=True)
                        wait_g(buf1, sg1, h_ref)
                        pltpu.async_copy(buf1, acc.at[adj_buf.at[j1]], ss1,
                                         add=True)
                        wait_s(buf0, ss0)

                        @pl.when(j0 + 2 < NB_E)
                        def _():
                            pltpu.async_copy(h_ref.at[src_buf.at[j0 + 2]],
                                             buf0, sg0)

                        wait_s(buf1, ss1)

                        @pl.when(j1 + 2 < NB_E)
                        def _():
                            pltpu.async_copy(h_ref.at[src_buf.at[j1 + 2]],
                                             buf1, sg1)

                        return carry

                    lax.fori_loop(0, NB_E // 2, eb, 0)
                    wait_g(buf0, sg0, h_ref)
                    pltpu.async_copy(buf0, acc.at[adj_buf.at[NB_E - 1]], ss0,
                                     add=True)
                    wait_s(buf0, ss0)
                    plsc.subcore_barrier()
                    dbase = s * DST
                    for i in range(DST // B_E):
                        pltpu.sync_copy(acc.at[pl.ds(dbase + i * B_E, B_E)],
                                        buf0)
                        pltpu.sync_copy(
                            buf0, o_ref.at[pl.ds(lo + dbase + i * B_E, B_E)])
                    plsc.subcore_barrier()

    return agg_kernel


@functools.lru_cache(maxsize=None)
def _build_b1():
    def body(deg_ref, x_ref, w_ref, h0, h1, h2, h3, ns_ref, nd_ref):
        d = deg_ref[...]
        ns = lax.rsqrt(jnp.maximum(d[:, 0:1], 1.0))
        nd = lax.rsqrt(jnp.maximum(d[:, 1:2], 1.0))
        h = jnp.dot(x_ref[...], w_ref[...],
                    preferred_element_type=jnp.float32) * ns
        for k, r in enumerate((h0, h1, h2, h3)):
            r[...] = h[:, k * 128:(k + 1) * 128]
        ns_ref[...] = ns
        nd_ref[...] = nd

    return pl.pallas_call(
        body,
        grid=(GRID,),
        in_specs=[
            pl.BlockSpec((N_BLK, 2), lambda i: (i, 0)),
            pl.BlockSpec((N_BLK, D_IN), lambda i: (i, 0)),
            pl.BlockSpec((D_IN, D_H), lambda i: (0, 0)),
        ],
        out_specs=[pl.BlockSpec((N_BLK, 128), lambda i: (i, 0))] * NCH
        + [pl.BlockSpec((N_BLK, 1), lambda i: (i, 0))] * 2,
        out_shape=[jax.ShapeDtypeStruct((N, 128), jnp.float32)] * NCH
        + [jax.ShapeDtypeStruct((N, 1), jnp.float32)] * 2,
    )


@functools.lru_cache(maxsize=None)
def _build_b23():
    def body(a0, a1, a2, a3, nd_ref, b_ref, ns_ref, w_ref, h0, h1, h2, h3):
        x = jnp.concatenate([a0[...], a1[...], a2[...], a3[...]], axis=1)
        x = jnp.maximum(x * nd_ref[...] + b_ref[...], 0.0)
        h = jnp.dot(x, w_ref[...],
                    preferred_element_type=jnp.float32) * ns_ref[...]
        for k, r in enumerate((h0, h1, h2, h3)):
            r[...] = h[:, k * 128:(k + 1) * 128]

    return pl.pallas_call(
        body,
        grid=(GRID,),
        in_specs=[pl.BlockSpec((N_BLK, 128), lambda i: (i, 0))] * NCH
        + [
            pl.BlockSpec((N_BLK, 1), lambda i: (i, 0)),
            pl.BlockSpec((1, D_H), lambda i: (0, 0)),
            pl.BlockSpec((N_BLK, 1), lambda i: (i, 0)),
            pl.BlockSpec((D_H, D_H), lambda i: (0, 0)),
        ],
        out_specs=[pl.BlockSpec((N_BLK, 128), lambda i: (i, 0))] * NCH,
        out_shape=[jax.ShapeDtypeStruct((N, 128), jnp.float32)] * NCH,
    )


@functools.lru_cache(maxsize=None)
def _build_b4():
    def body(a0, a1, a2, a3, nd_ref, b_ref, wm_ref, bm_ref, out_ref):
        x = jnp.concatenate([a0[...], a1[...], a2[...], a3[...]], axis=1)
        x = x * nd_ref[...] + b_ref[...]
        out_ref[...] = jnp.dot(x, wm_ref[...],
                               preferred_element_type=jnp.float32) + bm_ref[...]

    return pl.pallas_call(
        body,
        grid=(GRID,),
        in_specs=[pl.BlockSpec((N_BLK, 128), lambda i: (i, 0))] * NCH
        + [
            pl.BlockSpec((N_BLK, 1), lambda i: (i, 0)),
            pl.BlockSpec((1, D_H), lambda i: (0, 0)),
            pl.BlockSpec((D_H, D_OUT), lambda i: (0, 0)),
            pl.BlockSpec((1, D_OUT), lambda i: (0, 0)),
        ],
        out_specs=pl.BlockSpec((N_BLK, D_OUT), lambda i: (i, 0)),
        out_shape=jax.ShapeDtypeStruct((N, D_OUT), jnp.float32),
    )


def kernel(features, edge_index, W1, b1, W2, b2, W3, b3, Wm, bm):
    src = edge_index[0].reshape(NS, NB_E, B_E)
    dst = edge_index[1].reshape(NS, NB_E, B_E)
    eidx = edge_index.reshape(2, NS, EPT)

    deg_out, deg_in = _build_deg()(eidx)
    degT = jnp.stack([deg_out[:N], deg_in[:N]], axis=1)

    h0, h1, h2, h3, ns, nd = _build_b1()(degT, features, W1)
    agg = _build_agg()
    b23 = _build_b23()

    a = agg(h0, h1, h2, h3, src, dst)
    h = b23(*a, nd, b1.reshape(1, D_H), ns, W2)
    a = agg(*h, src, dst)
    h = b23(*a, nd, b2.reshape(1, D_H), ns, W3)
    a = agg(*h, src, dst)
    return _build_b4()(*a, nd, b3.reshape(1, D_H), Wm, bm.reshape(1, D_OUT))


# speedup vs baseline: 3.2793x; 1.0033x over previous
import functools

import jax
import jax.numpy as jnp
from jax import lax
from jax.experimental import pallas as pl
from jax.experimental.pallas import tpu as pltpu
from jax.experimental.pallas import tpu_sc as plsc

N = 10000
E = 160000
D_IN = 256
D_H = 512
D_OUT = 256

NS = 16
NC = 2
EPT = E // NS
B_E = 80
NB_E = EPT // B_E
N_PAD = 10240
STRIPE = N_PAD // NS
NCH = D_H // 128

HALF_N = 5120
ACC_R = 6400
ZST = ACC_R // NS
DST = HALF_N // NS

N_BLK = 1000
GRID = N // N_BLK


@functools.lru_cache(maxsize=None)
def _sc_mesh():
    return plsc.VectorSubcoreMesh(core_axis_name="c", subcore_axis_name="s")


_SC_PARAMS = pltpu.CompilerParams(needs_layout_passes=False)


@functools.lru_cache(maxsize=None)
def _build_deg():
    @functools.partial(
        pl.kernel,
        out_type=[jax.ShapeDtypeStruct((N_PAD,), jnp.float32)] * 2,
        mesh=_sc_mesh(),
        compiler_params=_SC_PARAMS,
        scratch_types=[
            pltpu.VMEM((EPT,), jnp.int32),
            pltpu.VMEM((N_PAD,), jnp.float32),
            pltpu.VMEM((NS, STRIPE), jnp.float32),
            pltpu.VMEM((STRIPE,), jnp.float32),
            pltpu.VMEM_SHARED((NS, N_PAD), jnp.float32),
        ],
    )
    def deg_kernel(eidx, out_src, out_dst, idx_buf, cnt, buf, stripe_buf,
                   shared):
        c = lax.axis_index("c")
        s = lax.axis_index("s")
        pltpu.sync_copy(eidx.at[c, s], idx_buf)
        zeros16 = jnp.zeros((16,), jnp.float32)

        def zero_body(i, carry):
            cnt[pl.ds(i * 16, 16)] = zeros16
            return carry

        lax.fori_loop(0, N_PAD // 16, zero_body, 0)
        ones16 = jnp.ones((16,), jnp.float32)

        def cnt_body(i, carry):
            v = idx_buf[pl.ds(i * 16, 16)]
            plsc.addupdate_scatter(cnt, [v], ones16)
            return carry

        lax.fori_loop(0, EPT // 16, cnt_body, 0)
        pltpu.sync_copy(cnt, shared.at[s])
        plsc.subcore_barrier()
        off = s * STRIPE
        pltpu.sync_copy(shared.at[:, pl.ds(off, STRIPE)], buf)

        def red_body(k, carry):
            acc = zeros16
            for t in range(NS):
                acc = acc + buf[t, pl.ds(k * 16, 16)]
            stripe_buf[pl.ds(k * 16, 16)] = acc
            return carry

        lax.fori_loop(0, STRIPE // 16, red_body, 0)

        @pl.when(c == 0)
        def _():
            pltpu.sync_copy(stripe_buf, out_src.at[pl.ds(off, STRIPE)])

        @pl.when(c == 1)
        def _():
            pltpu.sync_copy(stripe_buf, out_dst.at[pl.ds(off, STRIPE)])

    return deg_kernel


@functools.lru_cache(maxsize=None)
def _build_agg():
    @functools.partial(
        pl.kernel,
        out_type=[jax.ShapeDtypeStruct((N_PAD, 128), jnp.float32)] * NCH,
        mesh=_sc_mesh(),
        compiler_params=_SC_PARAMS,
        scratch_types=[
            pltpu.VMEM((NB_E, B_E), jnp.int32),
            pltpu.VMEM((NB_E, B_E), jnp.int32),
            pltpu.VMEM((NB_E, B_E), jnp.int32),
            pltpu.VMEM((B_E, 128), jnp.float32),
            pltpu.VMEM((B_E, 128), jnp.float32),
            pltpu.VMEM_SHARED((ACC_R, 128), jnp.float32),
            pltpu.SemaphoreType.DMA,
            pltpu.SemaphoreType.DMA,
            pltpu.SemaphoreType.DMA,
            pltpu.SemaphoreType.DMA,
        ],
    )
    def agg_kernel(h0, h1, h2, h3, srcr, dstr,
                   o0, o1, o2, o3,
                   src_buf, dst_buf, adj_buf, buf0, buf1, acc,
                   sg0, sg1, ss0, ss1):
        c = lax.axis_index("c")
        s = lax.axis_index("s")
        pltpu.sync_copy(srcr.at[s], src_buf)
        pltpu.sync_copy(dstr.at[s], dst_buf)
        zeros16 = jnp.zeros((16,), jnp.float32)
        h_refs = (h0, h1, h2, h3)
        o_refs = (o0, o1, o2, o3)

        def wait_g(buf, sem, h_ref):
            pltpu.make_async_copy(h_ref.at[src_buf.at[0]], buf, sem).wait()

        def wait_s(buf, sem):
            pltpu.make_async_copy(buf, acc.at[adj_buf.at[0]], sem).wait()

        for chunk in range(NCH):

            @pl.when(c == chunk // 2)
            def _(chunk=chunk):
                h_ref = h_refs[chunk]
                o_ref = o_refs[chunk]
                for p in range(2):
                    lo = p * HALF_N

                    def adj(r, carry):
                        for k in range(B_E // 16):
                            v = dst_buf[r, pl.ds(k * 16, 16)]
                            d = v - lo
                            ok = (d >= 0) & (d < HALF_N)
                            dump = HALF_N + (v & 1023)
                            adj_buf[r, pl.ds(k * 16, 16)] = jnp.where(
                                ok, d, dump)
                        return carry

                    lax.fori_loop(0, NB_E, adj, 0)

                    def zb(i, carry):
                        for k in range(128 // 16):
                            buf0[i, pl.ds(k * 16, 16)] = zeros16
                        return carry

                    lax.fori_loop(0, B_E, zb, 0)
                    zbase = s * ZST
                    for i in range(ZST // B_E):
                        pltpu.sync_copy(buf0,
                                        acc.at[pl.ds(zbase + i * B_E, B_E)])
                    plsc.subcore_barrier()

                    pltpu.async_copy(h_ref.at[src_buf.at[0]], buf0, sg0)
                    pltpu.async_copy(h_ref.at[src_buf.at[1]], buf1, sg1)

                    def eb(i, carry):
                        j0 = 2 * i
                        j1 = 2 * i + 1
                        wait_g(buf0, sg0, h_ref)
                        pltpu.async_copy(buf0, acc.at[adj_buf.at[j0]], ss0,
                                         add=True)
                        wait_g(buf1, sg1, h_ref)
                        pltpu.async_copy(buf1, acc.at[adj_buf.at[j1]], ss1,
                                         add=True)
                        wait_s(buf0, ss0)

                        @pl.when(j0 + 2 < NB_E)
                        def _():
                            pltpu.async_copy(h_ref.at[src_buf.at[j0 + 2]],
                                             buf0, sg0)

                        wait_s(buf1, ss1)

                        @pl.when(j1 + 2 < NB_E)
                        def _():
                            pltpu.async_copy(h_ref.at[src_buf.at[j1 + 2]],
                                             buf1, sg1)

                        return carry

                    lax.fori_loop(0, NB_E // 2, eb, 0)
                    wait_g(buf0, sg0, h_ref)
                    pltpu.async_copy(buf0, acc.at[adj_buf.at[NB_E - 1]], ss0,
                                     add=True)
                    wait_s(buf0, ss0)
                    plsc.subcore_barrier()
                    dbase = s * DST
                    for i in range(DST // B_E):
                        pltpu.sync_copy(acc.at[pl.ds(dbase + i * B_E, B_E)],
                                        buf0)
                        pltpu.sync_copy(
                            buf0, o_ref.at[pl.ds(lo + dbase + i * B_E, B_E)])
                    plsc.subcore_barrier()

    return agg_kernel


@functools.lru_cache(maxsize=None)
def _build_b1():
    def body(deg_ref, x_ref, w_ref, h0, h1, h2, h3, ns_ref, nd_ref):
        d = deg_ref[...]
        ns = lax.rsqrt(jnp.maximum(d[:, 0:1], 1.0))
        nd = lax.rsqrt(jnp.maximum(d[:, 1:2], 1.0))
        h = jnp.dot(x_ref[...], w_ref[...],
                    preferred_element_type=jnp.float32) * ns
        for k, r in enumerate((h0, h1, h2, h3)):
            r[...] = h[:, k * 128:(k + 1) * 128]
        ns_ref[...] = ns
        nd_ref[...] = nd

    return pl.pallas_call(
        body,
        grid=(GRID,),
        in_specs=[
            pl.BlockSpec((N_BLK, 2), lambda i: (i, 0)),
            pl.BlockSpec((N_BLK, D_IN), lambda i: (i, 0)),
            pl.BlockSpec((D_IN, D_H), lambda i: (0, 0)),
        ],
        out_specs=[pl.BlockSpec((N_BLK, 128), lambda i: (i, 0))] * NCH
        + [pl.BlockSpec((N_BLK, 1), lambda i: (i, 0))] * 2,
        out_shape=[jax.ShapeDtypeStruct((N, 128), jnp.float32)] * NCH
        + [jax.ShapeDtypeStruct((N, 1), jnp.float32)] * 2,
    )


@functools.lru_cache(maxsize=None)
def _build_b23():
    def body(a0, a1, a2, a3, nd_ref, b_ref, ns_ref, w_ref, h0, h1, h2, h3):
        x = jnp.concatenate([a0[...], a1[...], a2[...], a3[...]], axis=1)
        x = jnp.maximum(x * nd_ref[...] + b_ref[...], 0.0)
        h = jnp.dot(x, w_ref[...],
                    preferred_element_type=jnp.float32) * ns_ref[...]
        for k, r in enumerate((h0, h1, h2, h3)):
            r[...] = h[:, k * 128:(k + 1) * 128]

    return pl.pallas_call(
        body,
        grid=(GRID,),
        in_specs=[pl.BlockSpec((N_BLK, 128), lambda i: (i, 0))] * NCH
        + [
            pl.BlockSpec((N_BLK, 1), lambda i: (i, 0)),
            pl.BlockSpec((1, D_H), lambda i: (0, 0)),
            pl.BlockSpec((N_BLK, 1), lambda i: (i, 0)),
            pl.BlockSpec((D_H, D_H), lambda i: (0, 0)),
        ],
        out_specs=[pl.BlockSpec((N_BLK, 128), lambda i: (i, 0))] * NCH,
        out_shape=[jax.ShapeDtypeStruct((N, 128), jnp.float32)] * NCH,
    )


@functools.lru_cache(maxsize=None)
def _build_b4():
    def body(a0, a1, a2, a3, nd_ref, b_ref, wm_ref, bm_ref, out_ref):
        x = jnp.concatenate([a0[...], a1[...], a2[...], a3[...]], axis=1)
        x = x * nd_ref[...] + b_ref[...]
        out_ref[...] = jnp.dot(x, wm_ref[...],
                               preferred_element_type=jnp.float32) + bm_ref[...]

    return pl.pallas_call(
        body,
        grid=(GRID,),
        in_specs=[pl.BlockSpec((N_BLK, 128), lambda i: (i, 0))] * NCH
        + [
            pl.BlockSpec((N_BLK, 1), lambda i: (i, 0)),
            pl.BlockSpec((1, D_H), lambda i: (0, 0)),
            pl.BlockSpec((D_H, D_OUT), lambda i: (0, 0)),
            pl.BlockSpec((1, D_OUT), lambda i: (0, 0)),
        ],
        out_specs=pl.BlockSpec((N_BLK, D_OUT), lambda i: (i, 0)),
        out_shape=jax.ShapeDtypeStruct((N, D_OUT), jnp.float32),
    )


def kernel(features, edge_index, W1, b1, W2, b2, W3, b3, Wm, bm):
    src = edge_index[0].reshape(NS, NB_E, B_E)
    dst = edge_index[1].reshape(NS, NB_E, B_E)
    eidx = edge_index.reshape(2, NS, EPT)

    deg_out, deg_in = _build_deg()(eidx)
    degT = jnp.stack([deg_out[:N], deg_in[:N]], axis=1)

    h0, h1, h2, h3, ns, nd = _build_b1()(degT, features, W1)
    agg = _build_agg()
    b23 = _build_b23()

    a = agg(h0, h1, h2, h3, src, dst)
    h = b23(*a, nd, b1.reshape(1, D_H), ns, W2)
    a = agg(*h, src, dst)
    h = b23(*a, nd, b2.reshape(1, D_H), ns, W3)
    a = agg(*h, src, dst)
    return _build_b4()(*a, nd, b3.reshape(1, D_H), Wm, bm.reshape(1, D_OUT))


# speedup vs baseline: 4.9529x; 1.5104x over previous
import functools

import jax
import jax.numpy as jnp
from jax import lax
from jax.experimental import pallas as pl
from jax.experimental.pallas import tpu as pltpu
from jax.experimental.pallas import tpu_sc as plsc

N = 10000
E = 160000
D_IN = 256
D_H = 512
D_OUT = 256

NS = 16
NC = 2
EPT = E // NS
B_E = 80
NB_E = EPT // B_E
N_PAD = 10240
STRIPE = N_PAD // NS
NCH = D_H // 128

HALF_N = 5120
ACC_R = 5248
ZST = ACC_R // NS
DST = HALF_N // NS
CROWS = 126
TOT_PAD = CROWS * B_E

N_BLK = 1000
GRID = N // N_BLK


@functools.lru_cache(maxsize=None)
def _sc_mesh():
    return plsc.VectorSubcoreMesh(core_axis_name="c", subcore_axis_name="s")


_SC_PARAMS = pltpu.CompilerParams(needs_layout_passes=False)


@functools.lru_cache(maxsize=None)
def _build_deg():
    @functools.partial(
        pl.kernel,
        out_type=[jax.ShapeDtypeStruct((N_PAD,), jnp.float32)] * 2,
        mesh=_sc_mesh(),
        compiler_params=_SC_PARAMS,
        scratch_types=[
            pltpu.VMEM((EPT,), jnp.int32),
            pltpu.VMEM((N_PAD,), jnp.float32),
            pltpu.VMEM((NS, STRIPE), jnp.float32),
            pltpu.VMEM((STRIPE,), jnp.float32),
            pltpu.VMEM_SHARED((NS, N_PAD), jnp.float32),
        ],
    )
    def deg_kernel(eidx, out_src, out_dst, idx_buf, cnt, buf, stripe_buf,
                   shared):
        c = lax.axis_index("c")
        s = lax.axis_index("s")
        pltpu.sync_copy(eidx.at[c, s], idx_buf)
        zeros16 = jnp.zeros((16,), jnp.float32)

        def zero_body(i, carry):
            cnt[pl.ds(i * 16, 16)] = zeros16
            return carry

        lax.fori_loop(0, N_PAD // 16, zero_body, 0)
        ones16 = jnp.ones((16,), jnp.float32)

        def cnt_body(i, carry):
            v = idx_buf[pl.ds(i * 16, 16)]
            plsc.addupdate_scatter(cnt, [v], ones16)
            return carry

        lax.fori_loop(0, EPT // 16, cnt_body, 0)
        pltpu.sync_copy(cnt, shared.at[s])
        plsc.subcore_barrier()
        off = s * STRIPE
        pltpu.sync_copy(shared.at[:, pl.ds(off, STRIPE)], buf)

        def red_body(k, carry):
            acc = zeros16
            for t in range(NS):
                acc = acc + buf[t, pl.ds(k * 16, 16)]
            stripe_buf[pl.ds(k * 16, 16)] = acc
            return carry

        lax.fori_loop(0, STRIPE // 16, red_body, 0)

        @pl.when(c == 0)
        def _():
            pltpu.sync_copy(stripe_buf, out_src.at[pl.ds(off, STRIPE)])

        @pl.when(c == 1)
        def _():
            pltpu.sync_copy(stripe_buf, out_dst.at[pl.ds(off, STRIPE)])

    return deg_kernel


@functools.lru_cache(maxsize=None)
def _build_agg():
    @functools.partial(
        pl.kernel,
        out_type=[jax.ShapeDtypeStruct((N_PAD, 128), jnp.float32)] * NCH,
        mesh=_sc_mesh(),
        compiler_params=_SC_PARAMS,
        scratch_types=[
            pltpu.VMEM((CROWS, B_E), jnp.int32),
            pltpu.VMEM((CROWS, B_E), jnp.int32),
            pltpu.VMEM((CROWS, B_E), jnp.int32),
            pltpu.VMEM((CROWS, B_E), jnp.int32),
            pltpu.VMEM((B_E, 128), jnp.float32),
            pltpu.VMEM((B_E, 128), jnp.float32),
            pltpu.VMEM_SHARED((ACC_R, 128), jnp.float32),
            pltpu.SemaphoreType.DMA,
            pltpu.SemaphoreType.DMA,
            pltpu.SemaphoreType.DMA,
            pltpu.SemaphoreType.DMA,
        ],
    )
    def agg_kernel(h0, h1, h2, h3, srcr, dstr,
                   o0, o1, o2, o3,
                   cs1, cd1, cs0, cd0, buf0, buf1, acc,
                   sg0, sg1, ss0, ss1):
        c = lax.axis_index("c")
        s = lax.axis_index("s")
        pltpu.sync_copy(srcr.at[s], cs1.at[pl.ds(0, NB_E)])
        pltpu.sync_copy(dstr.at[s], cd1.at[pl.ds(0, NB_E)])
        zeros16 = jnp.zeros((16,), jnp.float32)
        iota16 = lax.iota(jnp.int32, 16)
        h_refs = (h0, h1, h2, h3)
        o_refs = (o0, o1, o2, o3)

        def cp(r, carry):
            m0, m1 = carry
            for k in range(B_E // 16):
                sv = cs1[r, pl.ds(k * 16, 16)]
                v = cd1[r, pl.ds(k * 16, 16)]
                ok0 = v < HALF_N
                k0 = ok0.astype(jnp.int32)
                k1 = 1 - k0
                pos0 = m0 + plsc.cumsum(k0) - k0
                pos1 = m1 + plsc.cumsum(k1) - k1
                plsc.store_scatter(cs0, [pos0 // B_E, pos0 % B_E], sv,
                                   mask=ok0)
                plsc.store_scatter(cd0, [pos0 // B_E, pos0 % B_E], v,
                                   mask=ok0)
                plsc.store_scatter(cs1, [pos1 // B_E, pos1 % B_E], sv,
                                   mask=~ok0)
                plsc.store_scatter(cd1, [pos1 // B_E, pos1 % B_E],
                                   v - HALF_N, mask=~ok0)
                m0 = m0 + jnp.sum(k0)
                m1 = m1 + jnp.sum(k1)
            return m0, m1

        m0, m1 = lax.fori_loop(0, NB_E, cp,
                               (jnp.int32(0), jnp.int32(0)))
        npairs = []
        for q, (csq, cdq, m) in enumerate(((cs0, cd0, m0), (cs1, cd1, m1))):
            np_q = (jnp.maximum(m, 1) + 2 * B_E - 1) // (2 * B_E)
            m_pad = np_q * (2 * B_E)
            for k in range(2 * B_E // 16):
                pos = m + iota16 + k * 16
                okp = pos < m_pad
                posc = jnp.minimum(pos, TOT_PAD - 1)
                sent_s = ((posc * 97) + s * 389) & 8191
                sent_d = HALF_N + ((posc + s * 61) & 127)
                plsc.store_scatter(csq, [posc // B_E, posc % B_E], sent_s,
                                   mask=okp)
                plsc.store_scatter(cdq, [posc // B_E, posc % B_E], sent_d,
                                   mask=okp)
            npairs.append(np_q)

        def wait_g(buf, sem, h_ref):
            pltpu.make_async_copy(h_ref.at[cs0.at[0]], buf, sem).wait()

        def wait_s(buf, sem):
            pltpu.make_async_copy(buf, acc.at[cd0.at[0]], sem).wait()

        for chunk in range(NCH):

            @pl.when(c == chunk // 2)
            def _(chunk=chunk):
                h_ref = h_refs[chunk]
                o_ref = o_refs[chunk]
                for p, (cs, cd) in enumerate(((cs0, cd0), (cs1, cd1))):
                    lo = p * HALF_N
                    npair = npairs[p]
                    nbatch = 2 * npair

                    def zb(i, carry):
                        for k in range(128 // 16):
                            buf0[i, pl.ds(k * 16, 16)] = zeros16
                        return carry

                    lax.fori_loop(0, B_E, zb, 0)
                    zbase = s * ZST
                    for i in range(ZST // B_E):
                        pltpu.sync_copy(buf0,
                                        acc.at[pl.ds(zbase + i * B_E, B_E)])
                    pltpu.sync_copy(
                        buf0.at[pl.ds(0, ZST % B_E)],
                        acc.at[pl.ds(zbase + (ZST // B_E) * B_E,
                                     ZST % B_E)])
                    plsc.subcore_barrier()

                    pltpu.async_copy(h_ref.at[cs.at[0]], buf0, sg0)
                    pltpu.async_copy(h_ref.at[cs.at[1]], buf1, sg1)

                    def eb(i, carry):
                        j0 = 2 * i
                        j1 = 2 * i + 1
                        wait_g(buf0, sg0, h_ref)
                        pltpu.async_copy(buf0, acc.at[cd.at[j0]], ss0,
                                         add=True)
                        wait_g(buf1, sg1, h_ref)
                        pltpu.async_copy(buf1, acc.at[cd.at[j1]], ss1,
                                         add=True)
                        wait_s(buf0, ss0)

                        @pl.when(j0 + 2 < nbatch)
                        def _():
                            pltpu.async_copy(h_ref.at[cs.at[j0 + 2]],
                                             buf0, sg0)

                        wait_s(buf1, ss1)

                        @pl.when(j1 + 2 < nbatch)
                        def _():
                            pltpu.async_copy(h_ref.at[cs.at[j1 + 2]],
                                             buf1, sg1)

                        return carry

                    lax.fori_loop(0, npair, eb, 0)
                    plsc.subcore_barrier()
                    dbase = s * DST
                    for i in range(DST // B_E):
                        pltpu.sync_copy(acc.at[pl.ds(dbase + i * B_E, B_E)],
                                        buf0)
                        pltpu.sync_copy(
                            buf0, o_ref.at[pl.ds(lo + dbase + i * B_E, B_E)])
                    plsc.subcore_barrier()

    return agg_kernel


@functools.lru_cache(maxsize=None)
def _build_b1():
    def body(deg_ref, x_ref, w_ref, h0, h1, h2, h3, ns_ref, nd_ref):
        d = deg_ref[...]
        ns = lax.rsqrt(jnp.maximum(d[:, 0:1], 1.0))
        nd = lax.rsqrt(jnp.maximum(d[:, 1:2], 1.0))
        h = jnp.dot(x_ref[...], w_ref[...],
                    preferred_element_type=jnp.float32) * ns
        for k, r in enumerate((h0, h1, h2, h3)):
            r[...] = h[:, k * 128:(k + 1) * 128]
        ns_ref[...] = ns
        nd_ref[...] = nd

    return pl.pallas_call(
        body,
        grid=(GRID,),
        in_specs=[
            pl.BlockSpec((N_BLK, 2), lambda i: (i, 0)),
            pl.BlockSpec((N_BLK, D_IN), lambda i: (i, 0)),
            pl.BlockSpec((D_IN, D_H), lambda i: (0, 0)),
        ],
        out_specs=[pl.BlockSpec((N_BLK, 128), lambda i: (i, 0))] * NCH
        + [pl.BlockSpec((N_BLK, 1), lambda i: (i, 0))] * 2,
        out_shape=[jax.ShapeDtypeStruct((N, 128), jnp.float32)] * NCH
        + [jax.ShapeDtypeStruct((N, 1), jnp.float32)] * 2,
    )


@functools.lru_cache(maxsize=None)
def _build_b23():
    def body(a0, a1, a2, a3, nd_ref, b_ref, ns_ref, w_ref, h0, h1, h2, h3):
        x = jnp.concatenate([a0[...], a1[...], a2[...], a3[...]], axis=1)
        x = jnp.maximum(x * nd_ref[...] + b_ref[...], 0.0)
        h = jnp.dot(x, w_ref[...],
                    preferred_element_type=jnp.float32) * ns_ref[...]
        for k, r in enumerate((h0, h1, h2, h3)):
            r[...] = h[:, k * 128:(k + 1) * 128]

    return pl.pallas_call(
        body,
        grid=(GRID,),
        in_specs=[pl.BlockSpec((N_BLK, 128), lambda i: (i, 0))] * NCH
        + [
            pl.BlockSpec((N_BLK, 1), lambda i: (i, 0)),
            pl.BlockSpec((1, D_H), lambda i: (0, 0)),
            pl.BlockSpec((N_BLK, 1), lambda i: (i, 0)),
            pl.BlockSpec((D_H, D_H), lambda i: (0, 0)),
        ],
        out_specs=[pl.BlockSpec((N_BLK, 128), lambda i: (i, 0))] * NCH,
        out_shape=[jax.ShapeDtypeStruct((N, 128), jnp.float32)] * NCH,
    )


@functools.lru_cache(maxsize=None)
def _build_b4():
    def body(a0, a1, a2, a3, nd_ref, b_ref, wm_ref, bm_ref, out_ref):
        x = jnp.concatenate([a0[...], a1[...], a2[...], a3[...]], axis=1)
        x = x * nd_ref[...] + b_ref[...]
        out_ref[...] = jnp.dot(x, wm_ref[...],
                               preferred_element_type=jnp.float32) + bm_ref[...]

    return pl.pallas_call(
        body,
        grid=(GRID,),
        in_specs=[pl.BlockSpec((N_BLK, 128), lambda i: (i, 0))] * NCH
        + [
            pl.BlockSpec((N_BLK, 1), lambda i: (i, 0)),
            pl.BlockSpec((1, D_H), lambda i: (0, 0)),
            pl.BlockSpec((D_H, D_OUT), lambda i: (0, 0)),
            pl.BlockSpec((1, D_OUT), lambda i: (0, 0)),
        ],
        out_specs=pl.BlockSpec((N_BLK, D_OUT), lambda i: (i, 0)),
        out_shape=jax.ShapeDtypeStruct((N, D_OUT), jnp.float32),
    )


def kernel(features, edge_index, W1, b1, W2, b2, W3, b3, Wm, bm):
    src = edge_index[0].reshape(NS, NB_E, B_E)
    dst = edge_index[1].reshape(NS, NB_E, B_E)
    eidx = edge_index.reshape(2, NS, EPT)

    deg_out, deg_in = _build_deg()(eidx)
    degT = jnp.stack([deg_out[:N], deg_in[:N]], axis=1)

    h0, h1, h2, h3, ns, nd = _build_b1()(degT, features, W1)
    agg = _build_agg()
    b23 = _build_b23()

    a = agg(h0, h1, h2, h3, src, dst)
    h = b23(*a, nd, b1.reshape(1, D_H), ns, W2)
    a = agg(*h, src, dst)
    h = b23(*a, nd, b2.reshape(1, D_H), ns, W3)
    a = agg(*h, src, dst)
    return _build_b4()(*a, nd, b3.reshape(1, D_H), Wm, bm.reshape(1, D_OUT))


# speedup vs baseline: 6.0680x; 1.2251x over previous
import functools

import jax
import jax.numpy as jnp
from jax import lax
from jax.experimental import pallas as pl
from jax.experimental.pallas import tpu as pltpu
from jax.experimental.pallas import tpu_sc as plsc

N = 10000
E = 160000
D_IN = 256
D_H = 512
D_OUT = 256

NS = 16
NC = 2
EPT = E // NS
B_E = 80
NB_E = EPT // B_E
N_PAD = 10240
STRIPE = N_PAD // NS
NCH = D_H // 128

HALF_N = 5120
ACC_R = 5248
ZST = ACC_R // NS
DST = HALF_N // NS
CROWS = 128
TOT_PAD = CROWS * B_E
NBUF = 4

N_BLK = 1000
GRID = N // N_BLK


@functools.lru_cache(maxsize=None)
def _sc_mesh():
    return plsc.VectorSubcoreMesh(core_axis_name="c", subcore_axis_name="s")


_SC_PARAMS = pltpu.CompilerParams(needs_layout_passes=False)


@functools.lru_cache(maxsize=None)
def _build_deg():
    @functools.partial(
        pl.kernel,
        out_type=[jax.ShapeDtypeStruct((N_PAD,), jnp.float32)] * 2,
        mesh=_sc_mesh(),
        compiler_params=_SC_PARAMS,
        scratch_types=[
            pltpu.VMEM((EPT,), jnp.int32),
            pltpu.VMEM((N_PAD,), jnp.float32),
            pltpu.VMEM((NS, STRIPE), jnp.float32),
            pltpu.VMEM((STRIPE,), jnp.float32),
            pltpu.VMEM_SHARED((NS, N_PAD), jnp.float32),
        ],
    )
    def deg_kernel(eidx, out_src, out_dst, idx_buf, cnt, buf, stripe_buf,
                   shared):
        c = lax.axis_index("c")
        s = lax.axis_index("s")
        pltpu.sync_copy(eidx.at[c, s], idx_buf)
        zeros16 = jnp.zeros((16,), jnp.float32)

        def zero_body(i, carry):
            cnt[pl.ds(i * 16, 16)] = zeros16
            return carry

        lax.fori_loop(0, N_PAD // 16, zero_body, 0)
        ones16 = jnp.ones((16,), jnp.float32)

        def cnt_body(i, carry):
            v = idx_buf[pl.ds(i * 16, 16)]
            plsc.addupdate_scatter(cnt, [v], ones16)
            return carry

        lax.fori_loop(0, EPT // 16, cnt_body, 0)
        pltpu.sync_copy(cnt, shared.at[s])
        plsc.subcore_barrier()
        off = s * STRIPE
        pltpu.sync_copy(shared.at[:, pl.ds(off, STRIPE)], buf)

        def red_body(k, carry):
            acc = zeros16
            for t in range(NS):
                acc = acc + buf[t, pl.ds(k * 16, 16)]
            stripe_buf[pl.ds(k * 16, 16)] = acc
            return carry

        lax.fori_loop(0, STRIPE // 16, red_body, 0)

        @pl.when(c == 0)
        def _():
            pltpu.sync_copy(stripe_buf, out_src.at[pl.ds(off, STRIPE)])

        @pl.when(c == 1)
        def _():
            pltpu.sync_copy(stripe_buf, out_dst.at[pl.ds(off, STRIPE)])

    return deg_kernel


@functools.lru_cache(maxsize=None)
def _build_agg():
    @functools.partial(
        pl.kernel,
        out_type=[jax.ShapeDtypeStruct((N_PAD, 128), jnp.float32)] * NCH,
        mesh=_sc_mesh(),
        compiler_params=_SC_PARAMS,
        scratch_types=[
            pltpu.VMEM((CROWS, B_E), jnp.int32),
            pltpu.VMEM((CROWS, B_E), jnp.int32),
            pltpu.VMEM((NBUF, B_E, 128), jnp.float32),
            pltpu.VMEM((NBUF, B_E), jnp.int32),
            pltpu.VMEM((NBUF, B_E), jnp.int32),
            pltpu.VMEM_SHARED((ACC_R, 128), jnp.float32),
            [pltpu.SemaphoreType.DMA] * NBUF,
            [pltpu.SemaphoreType.DMA] * NBUF,
        ],
    )
    def agg_kernel(h0, h1, h2, h3, srcr, dstr,
                   o0, o1, o2, o3,
                   p1, p0, bufs, st_s, st_d, acc, sgs, sss):
        c = lax.axis_index("c")
        s = lax.axis_index("s")
        pltpu.sync_copy(srcr.at[s], p1.at[pl.ds(0, NB_E)])
        pltpu.sync_copy(dstr.at[s], p0.at[pl.ds(0, NB_E)])
        zeros16 = jnp.zeros((16,), jnp.float32)
        iota16 = lax.iota(jnp.int32, 16)
        h_refs = (h0, h1, h2, h3)
        o_refs = (o0, o1, o2, o3)

        def cp(r, carry):
            m0, m1 = carry
            for k in range(B_E // 16):
                sv = p1[r, pl.ds(k * 16, 16)]
                v = p0[r, pl.ds(k * 16, 16)]
                ok0 = v < HALF_N
                k0 = ok0.astype(jnp.int32)
                k1 = 1 - k0
                pos0 = m0 + plsc.cumsum(k0) - k0
                pos1 = m1 + plsc.cumsum(k1) - k1
                pk0 = (sv << 13) | v
                pk1 = (sv << 13) | (v - HALF_N)
                plsc.store_scatter(p0, [pos0 // B_E, pos0 % B_E], pk0,
                                   mask=ok0)
                plsc.store_scatter(p1, [pos1 // B_E, pos1 % B_E], pk1,
                                   mask=~ok0)
                m0 = m0 + jnp.sum(k0)
                m1 = m1 + jnp.sum(k1)
            return m0, m1

        m0, m1 = lax.fori_loop(0, NB_E, cp,
                               (jnp.int32(0), jnp.int32(0)))
        ngroups = []
        for q, (pq, m) in enumerate(((p0, m0), (p1, m1))):
            ng_q = (jnp.maximum(m, 1) + NBUF * B_E - 1) // (NBUF * B_E)
            m_pad = ng_q * (NBUF * B_E)
            for k in range(NBUF * B_E // 16):
                pos = m + iota16 + k * 16
                okp = pos < m_pad
                posc = jnp.minimum(pos, TOT_PAD - 1)
                sent_s = ((posc * 97) + s * 389) & 8191
                sent_d = HALF_N + ((posc + s * 61) & 127)
                plsc.store_scatter(pq, [posc // B_E, posc % B_E],
                                   (sent_s << 13) | sent_d, mask=okp)
            ngroups.append(ng_q)

        def unpack(plist, j, q):
            for k in range(B_E // 16):
                w = plist[j, pl.ds(k * 16, 16)]
                st_s[q, pl.ds(k * 16, 16)] = lax.shift_right_logical(w, 13)
                st_d[q, pl.ds(k * 16, 16)] = w & 8191

        def wait_g(q):
            pltpu.make_async_copy(h0.at[st_s.at[0]], bufs.at[q],
                                  sgs[q]).wait()

        def wait_s(q):
            pltpu.make_async_copy(bufs.at[q], acc.at[st_d.at[0]],
                                  sss[q]).wait()

        for chunk in range(NCH):

            @pl.when(c == chunk // 2)
            def _(chunk=chunk):
                h_ref = h_refs[chunk]
                o_ref = o_refs[chunk]
                for p, plist in enumerate((p0, p1)):
                    lo = p * HALF_N
                    ngrp = ngroups[p]
                    nbatch = NBUF * ngrp

                    def zb(i, carry):
                        for k in range(128 // 16):
                            bufs[0, i, pl.ds(k * 16, 16)] = zeros16
                        return carry

                    lax.fori_loop(0, B_E, zb, 0)
                    zbase = s * ZST
                    for i in range(ZST // B_E):
                        pltpu.sync_copy(bufs.at[0],
                                        acc.at[pl.ds(zbase + i * B_E, B_E)])
                    pltpu.sync_copy(
                        bufs.at[0, pl.ds(0, ZST % B_E)],
                        acc.at[pl.ds(zbase + (ZST // B_E) * B_E,
                                     ZST % B_E)])
                    plsc.subcore_barrier()

                    for q in range(NBUF):
                        unpack(plist, q, q)
                        pltpu.async_copy(h_ref.at[st_s.at[q]], bufs.at[q],
                                         sgs[q])

                    def eb(i, carry):
                        for q in range(NBUF):
                            wait_g(q)
                            pltpu.async_copy(bufs.at[q], acc.at[st_d.at[q]],
                                             sss[q], add=True)
                        for q in range(NBUF):
                            j = NBUF * i + q
                            wait_s(q)

                            @pl.when(j + NBUF < nbatch)
                            def _(q=q, j=j):
                                unpack(plist, j + NBUF, q)
                                pltpu.async_copy(h_ref.at[st_s.at[q]],
                                                 bufs.at[q], sgs[q])

                        return carry

                    lax.fori_loop(0, ngrp, eb, 0)
                    plsc.subcore_barrier()
                    dbase = s * DST
                    for i in range(DST // B_E):
                        pltpu.sync_copy(acc.at[pl.ds(dbase + i * B_E, B_E)],
                                        bufs.at[0])
                        pltpu.sync_copy(
                            bufs.at[0],
                            o_ref.at[pl.ds(lo + dbase + i * B_E, B_E)])
                    plsc.subcore_barrier()

    return agg_kernel


@functools.lru_cache(maxsize=None)
def _build_b1():
    def body(deg_ref, x_ref, w_ref, h0, h1, h2, h3, ns_ref, nd_ref):
        d = deg_ref[...]
        ns = lax.rsqrt(jnp.maximum(d[:, 0:1], 1.0))
        nd = lax.rsqrt(jnp.maximum(d[:, 1:2], 1.0))
        h = jnp.dot(x_ref[...], w_ref[...],
                    preferred_element_type=jnp.float32) * ns
        for k, r in enumerate((h0, h1, h2, h3)):
            r[...] = h[:, k * 128:(k + 1) * 128]
        ns_ref[...] = ns
        nd_ref[...] = nd

    return pl.pallas_call(
        body,
        grid=(GRID,),
        in_specs=[
            pl.BlockSpec((N_BLK, 2), lambda i: (i, 0)),
            pl.BlockSpec((N_BLK, D_IN), lambda i: (i, 0)),
            pl.BlockSpec((D_IN, D_H), lambda i: (0, 0)),
        ],
        out_specs=[pl.BlockSpec((N_BLK, 128), lambda i: (i, 0))] * NCH
        + [pl.BlockSpec((N_BLK, 1), lambda i: (i, 0))] * 2,
        out_shape=[jax.ShapeDtypeStruct((N, 128), jnp.float32)] * NCH
        + [jax.ShapeDtypeStruct((N, 1), jnp.float32)] * 2,
    )


@functools.lru_cache(maxsize=None)
def _build_b23():
    def body(a0, a1, a2, a3, nd_ref, b_ref, ns_ref, w_ref, h0, h1, h2, h3):
        x = jnp.concatenate([a0[...], a1[...], a2[...], a3[...]], axis=1)
        x = jnp.maximum(x * nd_ref[...] + b_ref[...], 0.0)
        h = jnp.dot(x, w_ref[...],
                    preferred_element_type=jnp.float32) * ns_ref[...]
        for k, r in enumerate((h0, h1, h2, h3)):
            r[...] = h[:, k * 128:(k + 1) * 128]

    return pl.pallas_call(
        body,
        grid=(GRID,),
        in_specs=[pl.BlockSpec((N_BLK, 128), lambda i: (i, 0))] * NCH
        + [
            pl.BlockSpec((N_BLK, 1), lambda i: (i, 0)),
            pl.BlockSpec((1, D_H), lambda i: (0, 0)),
            pl.BlockSpec((N_BLK, 1), lambda i: (i, 0)),
            pl.BlockSpec((D_H, D_H), lambda i: (0, 0)),
        ],
        out_specs=[pl.BlockSpec((N_BLK, 128), lambda i: (i, 0))] * NCH,
        out_shape=[jax.ShapeDtypeStruct((N, 128), jnp.float32)] * NCH,
    )


@functools.lru_cache(maxsize=None)
def _build_b4():
    def body(a0, a1, a2, a3, nd_ref, b_ref, wm_ref, bm_ref, out_ref):
        x = jnp.concatenate([a0[...], a1[...], a2[...], a3[...]], axis=1)
        x = x * nd_ref[...] + b_ref[...]
        out_ref[...] = jnp.dot(x, wm_ref[...],
                               preferred_element_type=jnp.float32) + bm_ref[...]

    return pl.pallas_call(
        body,
        grid=(GRID,),
        in_specs=[pl.BlockSpec((N_BLK, 128), lambda i: (i, 0))] * NCH
        + [
            pl.BlockSpec((N_BLK, 1), lambda i: (i, 0)),
            pl.BlockSpec((1, D_H), lambda i: (0, 0)),
            pl.BlockSpec((D_H, D_OUT), lambda i: (0, 0)),
            pl.BlockSpec((1, D_OUT), lambda i: (0, 0)),
        ],
        out_specs=pl.BlockSpec((N_BLK, D_OUT), lambda i: (i, 0)),
        out_shape=jax.ShapeDtypeStruct((N, D_OUT), jnp.float32),
    )


def kernel(features, edge_index, W1, b1, W2, b2, W3, b3, Wm, bm):
    src = edge_index[0].reshape(NS, NB_E, B_E)
    dst = edge_index[1].reshape(NS, NB_E, B_E)
    eidx = edge_index.reshape(2, NS, EPT)

    deg_out, deg_in = _build_deg()(eidx)
    degT = jnp.stack([deg_out[:N], deg_in[:N]], axis=1)

    h0, h1, h2, h3, ns, nd = _build_b1()(degT, features, W1)
    agg = _build_agg()
    b23 = _build_b23()

    a = agg(h0, h1, h2, h3, src, dst)
    h = b23(*a, nd, b1.reshape(1, D_H), ns, W2)
    a = agg(*h, src, dst)
    h = b23(*a, nd, b2.reshape(1, D_H), ns, W3)
    a = agg(*h, src, dst)
    return _build_b4()(*a, nd, b3.reshape(1, D_H), Wm, bm.reshape(1, D_OUT))


# speedup vs baseline: 6.1204x; 1.0086x over previous
import functools

import jax
import jax.numpy as jnp
from jax import lax
from jax.experimental import pallas as pl
from jax.experimental.pallas import tpu as pltpu
from jax.experimental.pallas import tpu_sc as plsc

N = 10000
E = 160000
D_IN = 256
D_H = 512
D_OUT = 256

NS = 16
NC = 2
EPT = E // NS
B_E = 80
NB_E = EPT // B_E
N_PAD = 10240
STRIPE = N_PAD // NS
NCH = D_H // 128

HALF_N = 5120
ACC_R = 5248
ZST = ACC_R // NS
DST = HALF_N // NS
CROWS = 128
TOT_PAD = CROWS * B_E
NBUF = 5

N_BLK = 1000
GRID = N // N_BLK


@functools.lru_cache(maxsize=None)
def _sc_mesh():
    return plsc.VectorSubcoreMesh(core_axis_name="c", subcore_axis_name="s")


_SC_PARAMS = pltpu.CompilerParams(needs_layout_passes=False)


@functools.lru_cache(maxsize=None)
def _build_deg():
    @functools.partial(
        pl.kernel,
        out_type=[jax.ShapeDtypeStruct((N_PAD,), jnp.float32)] * 2,
        mesh=_sc_mesh(),
        compiler_params=_SC_PARAMS,
        scratch_types=[
            pltpu.VMEM((EPT,), jnp.int32),
            pltpu.VMEM((N_PAD,), jnp.float32),
            pltpu.VMEM((NS, STRIPE), jnp.float32),
            pltpu.VMEM((STRIPE,), jnp.float32),
            pltpu.VMEM_SHARED((NS, N_PAD), jnp.float32),
        ],
    )
    def deg_kernel(eidx, out_src, out_dst, idx_buf, cnt, buf, stripe_buf,
                   shared):
        c = lax.axis_index("c")
        s = lax.axis_index("s")
        pltpu.sync_copy(eidx.at[c, s], idx_buf)
        zeros16 = jnp.zeros((16,), jnp.float32)

        def zero_body(i, carry):
            cnt[pl.ds(i * 16, 16)] = zeros16
            return carry

        lax.fori_loop(0, N_PAD // 16, zero_body, 0)
        ones16 = jnp.ones((16,), jnp.float32)

        def cnt_body(i, carry):
            v = idx_buf[pl.ds(i * 16, 16)]
            plsc.addupdate_scatter(cnt, [v], ones16)
            return carry

        lax.fori_loop(0, EPT // 16, cnt_body, 0)
        pltpu.sync_copy(cnt, shared.at[s])
        plsc.subcore_barrier()
        off = s * STRIPE
        pltpu.sync_copy(shared.at[:, pl.ds(off, STRIPE)], buf)

        def red_body(k, carry):
            acc = zeros16
            for t in range(NS):
                acc = acc + buf[t, pl.ds(k * 16, 16)]
            stripe_buf[pl.ds(k * 16, 16)] = acc
            return carry

        lax.fori_loop(0, STRIPE // 16, red_body, 0)

        @pl.when(c == 0)
        def _():
            pltpu.sync_copy(stripe_buf, out_src.at[pl.ds(off, STRIPE)])

        @pl.when(c == 1)
        def _():
            pltpu.sync_copy(stripe_buf, out_dst.at[pl.ds(off, STRIPE)])

    return deg_kernel


@functools.lru_cache(maxsize=None)
def _build_agg():
    @functools.partial(
        pl.kernel,
        out_type=[jax.ShapeDtypeStruct((N_PAD, 128), jnp.float32)] * NCH,
        mesh=_sc_mesh(),
        compiler_params=_SC_PARAMS,
        scratch_types=[
            pltpu.VMEM((CROWS, B_E), jnp.int32),
            pltpu.VMEM((CROWS, B_E), jnp.int32),
            pltpu.VMEM((NBUF, B_E, 128), jnp.float32),
            pltpu.VMEM((NBUF, B_E), jnp.int32),
            pltpu.VMEM((NBUF, B_E), jnp.int32),
            pltpu.VMEM_SHARED((ACC_R, 128), jnp.float32),
            [pltpu.SemaphoreType.DMA] * NBUF,
            [pltpu.SemaphoreType.DMA] * NBUF,
        ],
    )
    def agg_kernel(h0, h1, h2, h3, srcr, dstr,
                   o0, o1, o2, o3,
                   p1, p0, bufs, st_s, st_d, acc, sgs, sss):
        c = lax.axis_index("c")
        s = lax.axis_index("s")
        pltpu.sync_copy(srcr.at[s], p1.at[pl.ds(0, NB_E)])
        pltpu.sync_copy(dstr.at[s], p0.at[pl.ds(0, NB_E)])
        zeros16 = jnp.zeros((16,), jnp.float32)
        iota16 = lax.iota(jnp.int32, 16)
        h_refs = (h0, h1, h2, h3)
        o_refs = (o0, o1, o2, o3)

        def cp(r, carry):
            m0, m1 = carry
            for k in range(B_E // 16):
                sv = p1[r, pl.ds(k * 16, 16)]
                v = p0[r, pl.ds(k * 16, 16)]
                ok0 = v < HALF_N
                k0 = ok0.astype(jnp.int32)
                k1 = 1 - k0
                pos0 = m0 + plsc.cumsum(k0) - k0
                pos1 = m1 + plsc.cumsum(k1) - k1
                pk0 = (sv << 13) | v
                pk1 = (sv << 13) | (v - HALF_N)
                plsc.store_scatter(p0, [pos0 // B_E, pos0 % B_E], pk0,
                                   mask=ok0)
                plsc.store_scatter(p1, [pos1 // B_E, pos1 % B_E], pk1,
                                   mask=~ok0)
                m0 = m0 + jnp.sum(k0)
                m1 = m1 + jnp.sum(k1)
            return m0, m1

        m0, m1 = lax.fori_loop(0, NB_E, cp,
                               (jnp.int32(0), jnp.int32(0)))
        ngroups = []
        for q, (pq, m) in enumerate(((p0, m0), (p1, m1))):
            ng_q = (jnp.maximum(m, 1) + NBUF * B_E - 1) // (NBUF * B_E)
            m_pad = ng_q * (NBUF * B_E)
            for k in range(NBUF * B_E // 16):
                pos = m + iota16 + k * 16
                okp = pos < m_pad
                posc = jnp.minimum(pos, TOT_PAD - 1)
                sent_s = ((posc * 97) + s * 389) & 8191
                sent_d = HALF_N + ((posc + s * 61) & 127)
                plsc.store_scatter(pq, [posc // B_E, posc % B_E],
                                   (sent_s << 13) | sent_d, mask=okp)
            ngroups.append(ng_q)

        def unpack(plist, j, q):
            for k in range(B_E // 16):
                w = plist[j, pl.ds(k * 16, 16)]
                st_s[q, pl.ds(k * 16, 16)] = lax.shift_right_logical(w, 13)
                st_d[q, pl.ds(k * 16, 16)] = w & 8191

        def wait_g(q):
            pltpu.make_async_copy(h0.at[st_s.at[0]], bufs.at[q],
                                  sgs[q]).wait()

        def wait_s(q):
            pltpu.make_async_copy(bufs.at[q], acc.at[st_d.at[0]],
                                  sss[q]).wait()

        for chunk in range(NCH):

            @pl.when(c == chunk // 2)
            def _(chunk=chunk):
                h_ref = h_refs[chunk]
                o_ref = o_refs[chunk]
                for p, plist in enumerate((p0, p1)):
                    lo = p * HALF_N
                    ngrp = ngroups[p]
                    nbatch = NBUF * ngrp

                    def zb(i, carry):
                        for k in range(128 // 16):
                            bufs[0, i, pl.ds(k * 16, 16)] = zeros16
                        return carry

                    lax.fori_loop(0, B_E, zb, 0)
                    zbase = s * ZST
                    for i in range(ZST // B_E):
                        pltpu.sync_copy(bufs.at[0],
                                        acc.at[pl.ds(zbase + i * B_E, B_E)])
                    pltpu.sync_copy(
                        bufs.at[0, pl.ds(0, ZST % B_E)],
                        acc.at[pl.ds(zbase + (ZST // B_E) * B_E,
                                     ZST % B_E)])
                    plsc.subcore_barrier()

                    for q in range(NBUF):
                        unpack(plist, q, q)
                        pltpu.async_copy(h_ref.at[st_s.at[q]], bufs.at[q],
                                         sgs[q])

                    def eb(i, carry):
                        for q in range(NBUF):
                            wait_g(q)
                            pltpu.async_copy(bufs.at[q], acc.at[st_d.at[q]],
                                             sss[q], add=True)
                        for q in range(NBUF):
                            j = NBUF * i + q
                            wait_s(q)

                            @pl.when(j + NBUF < nbatch)
                            def _(q=q, j=j):
                                unpack(plist, j + NBUF, q)
                                pltpu.async_copy(h_ref.at[st_s.at[q]],
                                                 bufs.at[q], sgs[q])

                        return carry

                    lax.fori_loop(0, ngrp, eb, 0)
                    plsc.subcore_barrier()
                    dbase = s * DST
                    for i in range(DST // B_E):
                        pltpu.sync_copy(acc.at[pl.ds(dbase + i * B_E, B_E)],
                                        bufs.at[0])
                        pltpu.sync_copy(
                            bufs.at[0],
                            o_ref.at[pl.ds(lo + dbase + i * B_E, B_E)])
                    plsc.subcore_barrier()

    return agg_kernel


@functools.lru_cache(maxsize=None)
def _build_b1():
    def body(deg_ref, x_ref, w_ref, h0, h1, h2, h3, ns_ref, nd_ref):
        d = deg_ref[...]
        ns = lax.rsqrt(jnp.maximum(d[:, 0:1], 1.0))
        nd = lax.rsqrt(jnp.maximum(d[:, 1:2], 1.0))
        h = jnp.dot(x_ref[...], w_ref[...],
                    preferred_element_type=jnp.float32) * ns
        for k, r in enumerate((h0, h1, h2, h3)):
            r[...] = h[:, k * 128:(k + 1) * 128]
        ns_ref[...] = ns
        nd_ref[...] = nd

    return pl.pallas_call(
        body,
        grid=(GRID,),
        in_specs=[
            pl.BlockSpec((N_BLK, 2), lambda i: (i, 0)),
            pl.BlockSpec((N_BLK, D_IN), lambda i: (i, 0)),
            pl.BlockSpec((D_IN, D_H), lambda i: (0, 0)),
        ],
        out_specs=[pl.BlockSpec((N_BLK, 128), lambda i: (i, 0))] * NCH
        + [pl.BlockSpec((N_BLK, 1), lambda i: (i, 0))] * 2,
        out_shape=[jax.ShapeDtypeStruct((N, 128), jnp.float32)] * NCH
        + [jax.ShapeDtypeStruct((N, 1), jnp.float32)] * 2,
    )


@functools.lru_cache(maxsize=None)
def _build_b23():
    def body(a0, a1, a2, a3, nd_ref, b_ref, ns_ref, w_ref, h0, h1, h2, h3):
        x = jnp.concatenate([a0[...], a1[...], a2[...], a3[...]], axis=1)
        x = jnp.maximum(x * nd_ref[...] + b_ref[...], 0.0)
        h = jnp.dot(x, w_ref[...],
                    preferred_element_type=jnp.float32) * ns_ref[...]
        for k, r in enumerate((h0, h1, h2, h3)):
            r[...] = h[:, k * 128:(k + 1) * 128]

    return pl.pallas_call(
        body,
        grid=(GRID,),
        in_specs=[pl.BlockSpec((N_BLK, 128), lambda i: (i, 0))] * NCH
        + [
            pl.BlockSpec((N_BLK, 1), lambda i: (i, 0)),
            pl.BlockSpec((1, D_H), lambda i: (0, 0)),
            pl.BlockSpec((N_BLK, 1), lambda i: (i, 0)),
            pl.BlockSpec((D_H, D_H), lambda i: (0, 0)),
        ],
        out_specs=[pl.BlockSpec((N_BLK, 128), lambda i: (i, 0))] * NCH,
        out_shape=[jax.ShapeDtypeStruct((N, 128), jnp.float32)] * NCH,
    )


@functools.lru_cache(maxsize=None)
def _build_b4():
    def body(a0, a1, a2, a3, nd_ref, b_ref, wm_ref, bm_ref, out_ref):
        x = jnp.concatenate([a0[...], a1[...], a2[...], a3[...]], axis=1)
        x = x * nd_ref[...] + b_ref[...]
        out_ref[...] = jnp.dot(x, wm_ref[...],
                               preferred_element_type=jnp.float32) + bm_ref[...]

    return pl.pallas_call(
        body,
        grid=(GRID,),
        in_specs=[pl.BlockSpec((N_BLK, 128), lambda i: (i, 0))] * NCH
        + [
            pl.BlockSpec((N_BLK, 1), lambda i: (i, 0)),
            pl.BlockSpec((1, D_H), lambda i: (0, 0)),
            pl.BlockSpec((D_H, D_OUT), lambda i: (0, 0)),
            pl.BlockSpec((1, D_OUT), lambda i: (0, 0)),
        ],
        out_specs=pl.BlockSpec((N_BLK, D_OUT), lambda i: (i, 0)),
        out_shape=jax.ShapeDtypeStruct((N, D_OUT), jnp.float32),
    )


def kernel(features, edge_index, W1, b1, W2, b2, W3, b3, Wm, bm):
    src = edge_index[0].reshape(NS, NB_E, B_E)
    dst = edge_index[1].reshape(NS, NB_E, B_E)
    eidx = edge_index.reshape(2, NS, EPT)

    deg_out, deg_in = _build_deg()(eidx)
    degT = jnp.stack([deg_out[:N], deg_in[:N]], axis=1)

    h0, h1, h2, h3, ns, nd = _build_b1()(degT, features, W1)
    agg = _build_agg()
    b23 = _build_b23()

    a = agg(h0, h1, h2, h3, src, dst)
    h = b23(*a, nd, b1.reshape(1, D_H), ns, W2)
    a = agg(*h, src, dst)
    h = b23(*a, nd, b2.reshape(1, D_H), ns, W3)
    a = agg(*h, src, dst)
    return _build_b4()(*a, nd, b3.reshape(1, D_H), Wm, bm.reshape(1, D_OUT))
